# Initial kernel scaffold; baseline (speedup 1.0000x reference)
import jax
import jax.numpy as jnp
from jax.experimental import pallas as pl


def kernel(feat, cu_seqlens, bracket, W1, b1, g1, be1, m1, v1, W2, b2, g2, be2, m2, v2, W3, b3):
    raise NotImplementedError("write your pallas kernel here")


# speedup vs baseline: 5.3422x; 5.3422x over previous
import jax
import jax.numpy as jnp
from jax import lax
from jax.experimental import pallas as pl
from jax.experimental.pallas import tpu as pltpu

N = 32768
B = 16
C = 96
RB = 4096
NBLK = N // RB
EPS = 1e-5


def _body(lo_ref, hi_ref, feat_ref, bracket_ref,
          W1_ref, b1_ref, g1_ref, be1_ref, m1_ref, v1_ref,
          W2_ref, b2_ref, g2_ref, be2_ref, m2_ref, v2_ref,
          W3_ref, b3_ref,
          pred_ref, loss_ref, cos_ref, acc_ref):
    i = pl.program_id(0)

    @pl.when(i == 0)
    def _():
        acc_ref[...] = jnp.zeros_like(acc_ref)

    idx = lax.broadcasted_iota(jnp.int32, (B, RB), 1) + i * RB
    lo = lo_ref[...]
    hi = hi_ref[...]
    mask = ((idx >= lo) & (idx < hi)).astype(jnp.float32)
    acc_ref[...] += jnp.dot(mask, feat_ref[...],
                            preferred_element_type=jnp.float32,
                            precision=lax.Precision.HIGHEST)

    @pl.when(i == NBLK - 1)
    def _():
        counts = jnp.maximum((hi - lo).astype(jnp.float32), 1.0)
        pooled = acc_ref[...] / counts
        h = jnp.dot(pooled, W1_ref[...], preferred_element_type=jnp.float32)
        h = h + b1_ref[...]
        h = g1_ref[...] * (h - m1_ref[...]) * lax.rsqrt(v1_ref[...] + EPS) \
            + be1_ref[...]
        h = jnp.maximum(h, 0.0)
        h = jnp.dot(h, W2_ref[...], preferred_element_type=jnp.float32)
        h = h + b2_ref[...]
        h = g2_ref[...] * (h - m2_ref[...]) * lax.rsqrt(v2_ref[...] + EPS) \
            + be2_ref[...]
        h = jnp.maximum(h, 0.0)
        pred = jnp.dot(h, W3_ref[...], preferred_element_type=jnp.float32)
        pred = pred + b3_ref[...]
        pred_ref[...] = pred
        target = bracket_ref[...]
        diff = pred - target
        loss_ref[...] = jnp.mean(diff * diff).reshape(1, 1)
        num = jnp.sum(pred * target, axis=1)
        den = (jnp.maximum(jnp.sqrt(jnp.sum(pred * pred, axis=1)), 1e-8)
               * jnp.maximum(jnp.sqrt(jnp.sum(target * target, axis=1)), 1e-8))
        cos_ref[...] = jnp.mean(num / den).reshape(1, 1)


def kernel(feat, cu_seqlens, bracket, W1, b1, g1, be1, m1, v1,
           W2, b2, g2, be2, m2, v2, W3, b3):
    lo = cu_seqlens[:-1].reshape(B, 1)
    hi = cu_seqlens[1:].reshape(B, 1)

    def whole(shape):
        return pl.BlockSpec(shape, lambda i: (0,) * len(shape))

    grid_spec = pltpu.PrefetchScalarGridSpec(
        num_scalar_prefetch=0,
        grid=(NBLK,),
        in_specs=[
            whole((B, 1)),
            whole((B, 1)),
            pl.BlockSpec((RB, C), lambda i: (i, 0)),
            whole((B, 3)),
            whole((C, 256)), whole((1, 256)), whole((1, 256)),
            whole((1, 256)), whole((1, 256)), whole((1, 256)),
            whole((256, 128)), whole((1, 128)), whole((1, 128)),
            whole((1, 128)), whole((1, 128)), whole((1, 128)),
            whole((128, 3)), whole((1, 3)),
        ],
        out_specs=[
            whole((B, 3)),
            whole((1, 1)),
            whole((1, 1)),
        ],
        scratch_shapes=[pltpu.VMEM((B, C), jnp.float32)],
    )

    pred, loss, cos = pl.pallas_call(
        _body,
        grid_spec=grid_spec,
        out_shape=[
            jax.ShapeDtypeStruct((B, 3), jnp.float32),
            jax.ShapeDtypeStruct((1, 1), jnp.float32),
            jax.ShapeDtypeStruct((1, 1), jnp.float32),
        ],
    )(lo, hi, feat, bracket,
      W1, b1.reshape(1, 256), g1.reshape(1, 256), be1.reshape(1, 256),
      m1.reshape(1, 256), v1.reshape(1, 256),
      W2, b2.reshape(1, 128), g2.reshape(1, 128), be2.reshape(1, 128),
      m2.reshape(1, 128), v2.reshape(1, 128),
      W3, b3.reshape(1, 3))
    return (pred, loss[0, 0], cos[0, 0])
